# Initial kernel scaffold; baseline (speedup 1.0000x reference)
import jax
import jax.numpy as jnp
from jax.experimental import pallas as pl


def kernel(visit_rel_times, time_embed_weight):
    raise NotImplementedError("write your pallas kernel here")


# speedup vs baseline: 17.1804x; 17.1804x over previous
import jax
import jax.numpy as jnp
from jax.experimental import pallas as pl

_BATCH_BLOCK = 32


def _tge_kernel(t_ref, w_ref, out_ref):
    t = t_ref[...][:, :, None]
    w0 = w_ref[0]
    w1 = w_ref[1]
    w2 = w_ref[2]
    w3 = w_ref[3]
    w4 = w_ref[4]
    out = jnp.where(
        t >= 48.0, w4,
        jnp.where(t >= 24.0, w3,
                  jnp.where(t >= 12.0, w2,
                            jnp.where(t >= 4.0, w1, w0))))
    out_ref[...] = out


def kernel(visit_rel_times, time_embed_weight):
    batch, hist = visit_rel_times.shape
    _, embed_dim = time_embed_weight.shape
    rb = _BATCH_BLOCK
    grid = (batch // rb,)
    return pl.pallas_call(
        _tge_kernel,
        grid=grid,
        in_specs=[
            pl.BlockSpec((rb, hist), lambda i: (i, 0)),
            pl.BlockSpec((5, embed_dim), lambda i: (0, 0)),
        ],
        out_specs=pl.BlockSpec((rb, hist, embed_dim), lambda i: (i, 0, 0)),
        out_shape=jax.ShapeDtypeStruct((batch, hist, embed_dim), jnp.float32),
    )(visit_rel_times, time_embed_weight)


# speedup vs baseline: 21.3747x; 1.2441x over previous
import jax
import jax.numpy as jnp
from jax.experimental import pallas as pl

_BATCH_BLOCK = 128


def _tge_kernel(t_ref, w_ref, out_ref):
    t = t_ref[...][:, :, None]
    w0 = w_ref[0]
    w1 = w_ref[1]
    w2 = w_ref[2]
    w3 = w_ref[3]
    w4 = w_ref[4]
    out = jnp.where(
        t >= 48.0, w4,
        jnp.where(t >= 24.0, w3,
                  jnp.where(t >= 12.0, w2,
                            jnp.where(t >= 4.0, w1, w0))))
    out_ref[...] = out


def kernel(visit_rel_times, time_embed_weight):
    batch, hist = visit_rel_times.shape
    _, embed_dim = time_embed_weight.shape
    rb = _BATCH_BLOCK
    grid = (batch // rb,)
    return pl.pallas_call(
        _tge_kernel,
        grid=grid,
        in_specs=[
            pl.BlockSpec((rb, hist), lambda i: (i, 0)),
            pl.BlockSpec((5, embed_dim), lambda i: (0, 0)),
        ],
        out_specs=pl.BlockSpec((rb, hist, embed_dim), lambda i: (i, 0, 0)),
        out_shape=jax.ShapeDtypeStruct((batch, hist, embed_dim), jnp.float32),
    )(visit_rel_times, time_embed_weight)
